# Initial kernel scaffold; baseline (speedup 1.0000x reference)
import jax
import jax.numpy as jnp
from jax.experimental import pallas as pl


def kernel(x, emb_0, emb_1, emb_2, emb_3, emb_4, emb_5, emb_6, emb_7, emb_8, emb_9, emb_10, emb_11, emb_12, emb_13, emb_14, emb_15, emb_16, W0, b0, W1, b1, W2, b2, W3, b3, g0, beta0, g1, beta1, g2, beta2):
    raise NotImplementedError("write your pallas kernel here")


# speedup vs baseline: 3.2379x; 3.2379x over previous
import functools

import numpy as np
import jax
import jax.numpy as jnp
from jax import lax
from jax.experimental import pallas as pl
from jax.experimental.pallas import tpu as pltpu
from jax.experimental.pallas import tpu_sc as plsc

_BINS = (512, 128, 256, 256, 64, 256, 256, 16, 256, 64, 16, 128, 64, 128, 64, 512, 512)
_EMB = 16
_NF = 17
_BATCH = 4096
_EPS = 1e-5
_OFFS = np.concatenate([[0], np.cumsum(_BINS)[:-1]]).astype(np.int32)
_VOCAB = int(np.sum(_BINS))

_NC, _NS = 2, 16
_NW = _NC * _NS
_B_TOT = _BATCH * _NF
_BPW = _B_TOT // _NW
_CHUNK = 128
_NCHUNK = _BPW // _CHUNK


def _gather_body(table_hbm, idx_hbm, out_hbm, idx_v, rows_v, sem):
    wid = lax.axis_index("s") * _NC + lax.axis_index("c")
    pltpu.sync_copy(idx_hbm.at[wid], idx_v)
    copies = [
        pltpu.async_copy(table_hbm.at[idx_v.at[j]], rows_v.at[j], sem)
        for j in range(_NCHUNK)
    ]
    for c in copies:
        c.wait()
    pltpu.sync_copy(rows_v, out_hbm.at[wid])


@functools.lru_cache(maxsize=None)
def _sc_gather():
    return pl.kernel(
        _gather_body,
        out_type=jax.ShapeDtypeStruct((_NW, _NCHUNK, _CHUNK, _EMB),
                                      jnp.float32),
        mesh=plsc.VectorSubcoreMesh(core_axis_name="c", subcore_axis_name="s",
                                    num_cores=_NC, num_subcores=_NS),
        scratch_types=[
            pltpu.VMEM((_NCHUNK, _CHUNK), jnp.int32),
            pltpu.VMEM((_NCHUNK, _CHUNK, _EMB), jnp.float32),
            pltpu.SemaphoreType.DMA,
        ],
        compiler_params=pltpu.CompilerParams(use_tc_tiling_on_sc=False),
    )


def _mlp_body(h_ref, w0, b0, g0, be0, w1, b1, g1, be1, w2, b2, g2, be2, w3, b3,
              out_ref):
    def layer(h, w, b, g, be):
        h = jnp.dot(h, w[:], preferred_element_type=jnp.float32) + b[:]
        h = jnp.maximum(h, 0.0)
        m = jnp.mean(h, axis=0, keepdims=True)
        v = jnp.mean((h - m) ** 2, axis=0, keepdims=True)
        return (h - m) * (g[:] * lax.rsqrt(v + _EPS)) + be[:]

    h = h_ref[:]
    h = layer(h, w0, b0, g0, be0)
    h = layer(h, w1, b1, g1, be1)
    h = layer(h, w2, b2, g2, be2)
    out_ref[:] = jnp.dot(h, w3[:], preferred_element_type=jnp.float32) + b3[:]


_mlp = pl.pallas_call(
    _mlp_body,
    out_shape=jax.ShapeDtypeStruct((_BATCH, 1), jnp.float32),
)


def kernel(x, emb_0, emb_1, emb_2, emb_3, emb_4, emb_5, emb_6, emb_7, emb_8,
           emb_9, emb_10, emb_11, emb_12, emb_13, emb_14, emb_15, emb_16,
           W0, b0, W1, b1, W2, b2, W3, b3, g0, beta0, g1, beta1, g2, beta2):
    embs = [emb_0, emb_1, emb_2, emb_3, emb_4, emb_5, emb_6, emb_7, emb_8,
            emb_9, emb_10, emb_11, emb_12, emb_13, emb_14, emb_15, emb_16]
    table = jnp.concatenate(embs, axis=0)
    flat_idx = (x + _OFFS[None, :]).reshape(_NW, _NCHUNK, _CHUNK)
    rows = _sc_gather()(table, flat_idx)
    h = rows.reshape(_BATCH, _NF * _EMB)
    r = lambda a: a.reshape(1, -1)
    out = _mlp(h, W0, r(b0), r(g0), r(beta0), W1, r(b1), r(g1), r(beta1),
               W2, r(b2), r(g2), r(beta2), W3, r(b3))
    return out
